# baseline (speedup 1.0000x reference)
import jax
import jax.numpy as jnp
from jax import lax
from jax.experimental import pallas as pl

B, S, E, H, C = 128, 16, 128, 128, 10.0
P = S + 1
NEG = -1e9


def _k1a_body(cc3, Wv, bv2, Wq, Wref, Whc, bhc2, iw2, vptr2, lout):
    cc0 = cc3[:, 0, :]
    cc2 = jnp.reshape(cc3[...], (B * S, E))
    ref2 = jnp.dot(cc2, Wref[...], preferred_element_type=jnp.float32)
    ref3 = jnp.reshape(ref2, (B, S, H))
    h_bar = jnp.dot(jnp.mean(cc3[...], axis=1), Whc[...],
                    preferred_element_type=jnp.float32) + bhc2[...]
    vcol = vptr2[...].reshape(H, 1)
    for p in range(P):
        if p < S:
            ch = jnp.concatenate([cc0, cc3[:, p, :]], axis=-1)
            qv = h_bar + jnp.dot(ch, Wv[...], preferred_element_type=jnp.float32) + bv2[...]
        else:
            qv = h_bar + jnp.dot(iw2[...], Wv[...], preferred_element_type=jnp.float32) + bv2[...]
        q = jnp.dot(qv, Wq[...], preferred_element_type=jnp.float32)
        t = jnp.tanh(q[:, None, :] + ref3)
        u = jnp.dot(t.reshape(B * S, H), vcol,
                    preferred_element_type=jnp.float32)
        lout[p] = C * jnp.tanh(u)


def _k1b_body(node3, Wq_l, Wref_l, vl2, out):
    nd = node3[...]
    cb = nd.shape[0]
    mn = jnp.mean(nd, axis=1)
    q_l = jnp.dot(mn, Wq_l[...], preferred_element_type=jnp.float32)
    ref2 = jnp.dot(jnp.reshape(nd, (cb * S, E)), Wref_l[...],
                   preferred_element_type=jnp.float32)
    ref3 = jnp.reshape(ref2, (cb, S, H))
    t = jnp.tanh(q_l[:, None, :] + ref3)
    u = jnp.dot(t.reshape(cb * S, H), vl2[...].reshape(H, 1),
                preferred_element_type=jnp.float32)
    out[...] = C * jnp.tanh(u)


def _k2_body(Lp3, la3, lm3, ox3, oy3, hm2, on3, ln3, jn3,
             clp, nlp, crw, nrw, ca, na):
    iota_l = lax.broadcasted_iota(jnp.int32, (B, S), 1)
    iota_p = lax.broadcasted_iota(jnp.int32, (P, B, 1), 0)
    Lp = Lp3[...]

    def step(i, carry):
        prev_oh, hm, ca_a, lp_a, ohall = carry
        L = jnp.sum(prev_oh * Lp, axis=0)
        masked = jnp.where(hm == 1.0, NEG, L)
        y = masked + on3[pl.ds(i, 1)][0]
        ymax = jnp.max(y, axis=1, keepdims=True)
        cand = jnp.where(y == ymax, iota_l, S)
        idxr = jnp.min(cand, axis=1, keepdims=True)
        idxr = jnp.where(i == 0, 0, idxr)
        oh = (iota_l == idxr).astype(jnp.float32)
        m2 = jnp.max(masked, axis=1, keepdims=True)
        se = jnp.sum(jnp.exp(masked - m2), axis=1, keepdims=True)
        sel = jnp.sum(oh * masked, axis=1, keepdims=True)
        lp_row = sel - m2 - jnp.log(se)
        hm = jnp.maximum(hm, oh)
        ca_a = jnp.where(iota_l == i, idxr, ca_a)
        lp_a = jnp.where(iota_l == i, lp_row, lp_a)
        io_i = lax.broadcasted_iota(jnp.int32, (S, B, S), 0)
        ohall = jnp.where(io_i == i, oh[None], ohall)
        prev_oh = (iota_p == idxr).astype(jnp.float32)
        return prev_oh, hm, ca_a, lp_a, ohall

    prev0 = (iota_p == S).astype(jnp.float32)
    carry = (prev0, hm2[...], jnp.zeros((B, S), jnp.int32),
             jnp.zeros((B, S), jnp.float32), jnp.zeros((S, B, S), jnp.float32))
    prev_oh, hm, ca_a, lp_a, ohall = lax.fori_loop(0, S, step, carry)

    ca[...] = ca_a
    clp[...] = jnp.sum(lp_a, axis=1, keepdims=True)

    oh4 = ohall[:, :, :, None]
    Lg = jnp.sum(oh4 * la3[...][None], axis=2)
    lmg = jnp.sum(oh4 * lm3[...][None], axis=2)
    oxg = jnp.sum(oh4 * ox3[...][None], axis=2)
    oyg = jnp.sum(oh4 * oy3[...][None], axis=2)
    ml = jnp.where(lmg == 1.0, NEG, Lg)

    io2 = lax.broadcasted_iota(jnp.int32, (S, B, S), 2)
    y2 = ml + ln3[...]
    mm = jnp.max(y2, axis=2, keepdims=True)
    cand2 = jnp.where(y2 == mm, io2, S)
    laidx = jnp.min(cand2, axis=2, keepdims=True)
    laoh = (io2 == laidx).astype(jnp.float32)
    lastx = jnp.sum(laoh * oxg, axis=2)
    lasty = jnp.sum(laoh * oyg, axis=2)
    initx = oxg[:, :, 0]
    inity = oyg[:, :, 0]
    dx = lastx[: S - 1] - initx[1:]
    dy = lasty[: S - 1] - inity[1:]
    crw[...] = jnp.sum(jnp.sqrt(dx * dx + dy * dy + 1e-12), axis=0,
                       keepdims=True)

    ml15 = ml[S - 1]
    io3 = lax.broadcasted_iota(jnp.int32, (B, S, S), 2)
    y3 = ml15[:, None, :] + jn3[...]
    m3j = jnp.max(y3, axis=2, keepdims=True)
    cand3 = jnp.where(y3 == m3j, io3, S)
    ljidx = jnp.min(cand3, axis=2, keepdims=True)
    ljoh = (io3 == ljidx).astype(jnp.float32)
    na[...] = ljidx[:, :, 0]
    m3 = jnp.max(ml15, axis=1, keepdims=True)
    lse = m3 + jnp.log(jnp.sum(jnp.exp(ml15 - m3), axis=1, keepdims=True))
    selj = jnp.sum(ljoh * ml15[:, None, :], axis=2)
    nlp[...] = jnp.sum(selj - lse, axis=1, keepdims=True)
    ox15 = oxg[S - 1]
    oy15 = oyg[S - 1]
    lxj = jnp.sum(ljoh * ox15[:, None, :], axis=2)
    lyj = jnp.sum(ljoh * oy15[:, None, :], axis=2)
    ix = ox15[:, 0:1]
    iy = oy15[:, 0:1]
    rwj = jnp.sqrt((lxj - ix) ** 2 + (lyj - iy) ** 2 + 1e-12)
    nrw[...] = jnp.sum(rwj, axis=1, keepdims=True)


def _gumbel_stream(ids, out_axes=0):
    skey = jax.random.key(42)

    def one(i):
        u = jax.random.uniform(jax.random.fold_in(skey, i), (B, S),
                               minval=1e-6, maxval=1.0 - 1e-6)
        return -jnp.log(-jnp.log(u))

    return jax.vmap(one, out_axes=out_axes)(ids)


@jax.jit
def kernel(node_context, original_data, cell_context, high_mask, low_mask,
           init_w, Whc, bhc, Wv, bv, Wq, Wref, vptr, Wq_l, Wref_l, v_l):
    f32 = jnp.float32

    onoise = _gumbel_stream(jnp.arange(S))
    lnoise = _gumbel_stream((jnp.arange(S) + 1) * 1000 + (S - 1))
    jnoise = _gumbel_stream(S * 1000 + jnp.arange(S), out_axes=1)

    full = lambda shp: pl.BlockSpec(shp, lambda *_: (0,) * len(shp))
    r2 = lambda a: a.reshape(1, -1)

    lout = pl.pallas_call(
        _k1a_body,
        grid=(1,),
        in_specs=[full((B, S, E)), full((2 * E, E)), full((1, E)),
                  full((E, H)), full((E, H)), full((E, E)), full((1, E)),
                  full((1, 2 * E)), full((1, H))],
        out_specs=full((P, B * S, 1)),
        out_shape=jax.ShapeDtypeStruct((P, B * S, 1), f32),
    )(cell_context, Wv, r2(bv), Wq, Wref, Whc, r2(bhc), r2(init_w), r2(vptr))

    CB = 256
    node3 = node_context.reshape(B * S, S, E)
    logits2 = pl.pallas_call(
        _k1b_body,
        grid=(B * S // CB,),
        in_specs=[pl.BlockSpec((CB, S, E), lambda i: (i, 0, 0)),
                  full((E, H)), full((E, H)), full((1, H))],
        out_specs=pl.BlockSpec((CB * S, 1), lambda i: (i, 0)),
        out_shape=jax.ShapeDtypeStruct((B * S * S, 1), f32),
    )(node3, Wq_l, Wref_l, r2(v_l))

    outs = pl.pallas_call(
        _k2_body,
        grid=(1,),
        in_specs=[full((P, B, S)), full((B, S, S)), full((B, S, S)),
                  full((B, S, S)), full((B, S, S)), full((B, S)),
                  full((S, B, S)), full((S, B, S)), full((B, S, S))],
        out_specs=[full((B, 1)), full((B, 1)), full((1, B)), full((B, 1)),
                   full((B, S)), full((B, S))],
        out_shape=[jax.ShapeDtypeStruct((B, 1), f32),
                   jax.ShapeDtypeStruct((B, 1), f32),
                   jax.ShapeDtypeStruct((1, B), f32),
                   jax.ShapeDtypeStruct((B, 1), f32),
                   jax.ShapeDtypeStruct((B, S), jnp.int32),
                   jax.ShapeDtypeStruct((B, S), jnp.int32)],
    )(lout.reshape(P, B, S), logits2.reshape(B, S, S), low_mask,
      original_data[..., 0], original_data[..., 1], high_mask,
      onoise, lnoise, jnoise)

    clp, nlp, crw, nrw, ca, na = outs
    return (clp.reshape(B), nlp.reshape(B), crw.reshape(B), nrw.reshape(B),
            ca, na)


# speedup vs baseline: 2.0199x; 2.0199x over previous
import jax
import jax.numpy as jnp
from jax import lax
from jax.experimental import pallas as pl

B, S, E, H, C = 128, 16, 128, 128, 10.0
P = S + 1
NEG = -1e9


def _k1a_body(cc3, Wv, bv2, Wq, Wref, Whc, bhc2, iw2, vptr2, lout):
    cc0 = cc3[:, 0, :]
    cc2 = jnp.reshape(cc3[...], (B * S, E))
    ref2 = jnp.dot(cc2, Wref[...], preferred_element_type=jnp.float32)
    ref3 = jnp.reshape(ref2, (B, S, H))
    h_bar = jnp.dot(jnp.mean(cc3[...], axis=1), Whc[...],
                    preferred_element_type=jnp.float32) + bhc2[...]
    vcol = vptr2[...].reshape(H, 1)
    for p in range(P):
        if p < S:
            ch = jnp.concatenate([cc0, cc3[:, p, :]], axis=-1)
            qv = h_bar + jnp.dot(ch, Wv[...], preferred_element_type=jnp.float32) + bv2[...]
        else:
            qv = h_bar + jnp.dot(iw2[...], Wv[...], preferred_element_type=jnp.float32) + bv2[...]
        q = jnp.dot(qv, Wq[...], preferred_element_type=jnp.float32)
        t = jnp.tanh(q[:, None, :] + ref3)
        u = jnp.dot(t.reshape(B * S, H), vcol,
                    preferred_element_type=jnp.float32)
        lout[p] = C * jnp.tanh(u)


def _k1b_body(node3, Wq_l, Wref_l, vl2, out):
    nd = node3[...]
    cb = nd.shape[0]
    mn = jnp.mean(nd, axis=1)
    q_l = jnp.dot(mn, Wq_l[...], preferred_element_type=jnp.float32)
    ref2 = jnp.dot(jnp.reshape(nd, (cb * S, E)), Wref_l[...],
                   preferred_element_type=jnp.float32)
    ref3 = jnp.reshape(ref2, (cb, S, H))
    t = jnp.tanh(q_l[:, None, :] + ref3)
    u = jnp.dot(t.reshape(cb * S, H), vl2[...].reshape(H, 1),
                preferred_element_type=jnp.float32)
    out[...] = C * jnp.tanh(u)


def _k2_body(LpT, laT, lmT, oxT, oyT, hmT, nsT,
             clp, nlp, crw, nrw, caT, naT):
    iota_s = lax.broadcasted_iota(jnp.int32, (S, B), 0)
    iota_p = lax.broadcasted_iota(jnp.int32, (P, 1, B), 0)
    Lp = LpT[...]

    def step(i, carry):
        prev_oh, hm, ca, lp, ohall = carry
        L = jnp.sum(prev_oh * Lp, axis=0)
        masked = jnp.where(hm == 1.0, NEG, L)
        noise = nsT[pl.ds(i, 1)][0]
        y = masked + noise
        ymax = jnp.max(y, axis=0, keepdims=True)
        cand = jnp.where(y == ymax, iota_s, S)
        idxr = jnp.min(cand, axis=0, keepdims=True)
        idxr = jnp.where(i == 0, 0, idxr)
        oh = (iota_s == idxr).astype(jnp.float32)
        m2 = jnp.max(masked, axis=0, keepdims=True)
        se = jnp.sum(jnp.exp(masked - m2), axis=0, keepdims=True)
        sel = jnp.sum(oh * masked, axis=0, keepdims=True)
        lp_row = sel - m2 - jnp.log(se)
        hm = jnp.maximum(hm, oh)
        ca = jnp.where(iota_s == i, idxr, ca)
        lp = jnp.where(iota_s == i, lp_row, lp)
        io_i = lax.broadcasted_iota(jnp.int32, (S, 1, B), 0)
        ohall = jnp.where(io_i == i, oh[None, :, :], ohall)
        prev_oh = (iota_p == idxr).astype(jnp.float32)
        return prev_oh, hm, ca, lp, ohall

    prev0 = (iota_p == S).astype(jnp.float32)
    carry = (prev0, hmT[...], jnp.zeros((S, B), jnp.int32),
             jnp.zeros((S, B), jnp.float32), jnp.zeros((S, S, B), jnp.float32))
    prev_oh, hm, ca, lp, ohall = lax.fori_loop(0, S, step, carry)

    caT[...] = ca
    clp[...] = jnp.sum(lp, axis=0, keepdims=True)

    oh4 = ohall[:, :, None, :]
    Lg = jnp.sum(oh4 * laT[...][None], axis=1)
    lmg = jnp.sum(oh4 * lmT[...][None], axis=1)
    oxg = jnp.sum(oh4 * oxT[...][None], axis=1)
    oyg = jnp.sum(oh4 * oyT[...][None], axis=1)
    ml = jnp.where(lmg == 1.0, NEG, Lg)

    io1 = lax.broadcasted_iota(jnp.int32, (S, S, B), 1)
    y2 = ml + nsT[pl.ds(S, S)]
    m = jnp.max(y2, axis=1, keepdims=True)
    cand = jnp.where(y2 == m, io1, S)
    laidx = jnp.min(cand, axis=1, keepdims=True)
    laoh = (io1 == laidx).astype(jnp.float32)
    lastx = jnp.sum(laoh * oxg, axis=1)
    lasty = jnp.sum(laoh * oyg, axis=1)
    initx = oxg[:, 0, :]
    inity = oyg[:, 0, :]
    dx = lastx[: S - 1] - initx[1:]
    dy = lasty[: S - 1] - inity[1:]
    crw[...] = jnp.sum(jnp.sqrt(dx * dx + dy * dy + 1e-12), axis=0,
                       keepdims=True)

    ml15 = ml[S - 1]
    y3 = ml15[None] + nsT[pl.ds(2 * S, S)]
    m3j = jnp.max(y3, axis=1, keepdims=True)
    cand3 = jnp.where(y3 == m3j, io1, S)
    ljidx = jnp.min(cand3, axis=1, keepdims=True)
    ljoh = (io1 == ljidx).astype(jnp.float32)
    naT[...] = ljidx[:, 0, :]
    m3 = jnp.max(ml15, axis=0, keepdims=True)
    lse = m3 + jnp.log(jnp.sum(jnp.exp(ml15 - m3), axis=0, keepdims=True))
    selj = jnp.sum(ljoh * ml15[None], axis=1)
    nlp[...] = jnp.sum(selj - lse, axis=0, keepdims=True)
    ox15 = oxg[S - 1]
    oy15 = oyg[S - 1]
    lxj = jnp.sum(ljoh * ox15[None], axis=1)
    lyj = jnp.sum(ljoh * oy15[None], axis=1)
    ix = ox15[0:1, :]
    iy = oy15[0:1, :]
    rwj = jnp.sqrt((lxj - ix) ** 2 + (lyj - iy) ** 2 + 1e-12)
    nrw[...] = jnp.sum(rwj, axis=0, keepdims=True)


def _gumbel_stream(ids):
    skey = jax.random.key(42)

    def one(i):
        u = jax.random.uniform(jax.random.fold_in(skey, i), (B, S),
                               minval=1e-6, maxval=1.0 - 1e-6)
        return -jnp.log(-jnp.log(u))

    return jax.vmap(one)(ids)


@jax.jit
def kernel(node_context, original_data, cell_context, high_mask, low_mask,
           init_w, Whc, bhc, Wv, bv, Wq, Wref, vptr, Wq_l, Wref_l, v_l):
    f32 = jnp.float32

    ids = jnp.concatenate([jnp.arange(S),
                           (jnp.arange(S) + 1) * 1000 + (S - 1),
                           S * 1000 + jnp.arange(S)])
    noise = _gumbel_stream(ids)
    nsT = jnp.transpose(noise, (0, 2, 1))

    full = lambda shp: pl.BlockSpec(shp, lambda *_: (0,) * len(shp))
    r2 = lambda a: a.reshape(1, -1)

    lout = pl.pallas_call(
        _k1a_body,
        grid=(1,),
        in_specs=[full((B, S, E)), full((2 * E, E)), full((1, E)),
                  full((E, H)), full((E, H)), full((E, E)), full((1, E)),
                  full((1, 2 * E)), full((1, H))],
        out_specs=full((P, B * S, 1)),
        out_shape=jax.ShapeDtypeStruct((P, B * S, 1), f32),
    )(cell_context, Wv, r2(bv), Wq, Wref, Whc, r2(bhc), r2(init_w), r2(vptr))
    lout = lout.reshape(P, B, S)

    CB = 256
    node3 = node_context.reshape(B * S, S, E)
    logits2 = pl.pallas_call(
        _k1b_body,
        grid=(B * S // CB,),
        in_specs=[pl.BlockSpec((CB, S, E), lambda i: (i, 0, 0)),
                  full((E, H)), full((E, H)), full((1, H))],
        out_specs=pl.BlockSpec((CB * S, 1), lambda i: (i, 0)),
        out_shape=jax.ShapeDtypeStruct((B * S * S, 1), f32),
    )(node3, Wq_l, Wref_l, r2(v_l))

    laT = jnp.transpose(logits2.reshape(B, S, S), (1, 2, 0))
    lmT = jnp.transpose(low_mask, (1, 2, 0))
    oxT = jnp.transpose(original_data[..., 0], (1, 2, 0))
    oyT = jnp.transpose(original_data[..., 1], (1, 2, 0))
    LpT = jnp.transpose(lout, (0, 2, 1))
    hmT = jnp.transpose(high_mask, (1, 0))

    outs = pl.pallas_call(
        _k2_body,
        grid=(1,),
        in_specs=[full((P, S, B)), full((S, S, B)), full((S, S, B)),
                  full((S, S, B)), full((S, S, B)), full((S, B)),
                  full((3 * S, S, B))],
        out_specs=[full((1, B)), full((1, B)), full((1, B)), full((1, B)),
                   full((S, B)), full((S, B))],
        out_shape=[jax.ShapeDtypeStruct((1, B), f32),
                   jax.ShapeDtypeStruct((1, B), f32),
                   jax.ShapeDtypeStruct((1, B), f32),
                   jax.ShapeDtypeStruct((1, B), f32),
                   jax.ShapeDtypeStruct((S, B), jnp.int32),
                   jax.ShapeDtypeStruct((S, B), jnp.int32)],
    )(LpT, laT, lmT, oxT, oyT, hmT, nsT)

    clp, nlp, crw, nrw, caT, naT = outs
    return (clp.reshape(B), nlp.reshape(B), crw.reshape(B), nrw.reshape(B),
            jnp.transpose(caT, (1, 0)), jnp.transpose(naT, (1, 0)))
